# Initial kernel scaffold; baseline (speedup 1.0000x reference)
import jax
import jax.numpy as jnp
from jax.experimental import pallas as pl


def kernel(bbs_inf, phi_W, psi_W, omega_W):
    raise NotImplementedError("write your pallas kernel here")


# speedup vs baseline: 1.8966x; 1.8966x over previous
import functools

import jax
import jax.numpy as jnp
from jax import lax
from jax.experimental import pallas as pl
from jax.experimental.pallas import tpu as pltpu
from jax.experimental.pallas import tpu_sc as plsc

_LANES = 16
_GATHER = 128


def kernel(bbs_inf, phi_W, psi_W, omega_W):
    B, L, T = bbs_inf.shape
    V, D = phi_W.shape
    R = B * L
    N = R * T

    table = jnp.concatenate([phi_W, psi_W, omega_W], axis=0)
    idx_flat = bbs_inf.reshape(N)

    info = plsc.get_sparse_core_info()
    NW = info.num_cores * info.num_subcores
    per_w = N // NW
    CH = 3 * _GATHER
    n_chunks = per_w // CH

    mesh = plsc.VectorSubcoreMesh(core_axis_name="c", subcore_axis_name="s")

    @functools.partial(
        pl.kernel,
        mesh=mesh,
        out_type=jax.ShapeDtypeStruct((N, D), jnp.float32),
        scratch_types=[
            pltpu.VMEM((CH,), jnp.int32),
            pltpu.VMEM((T, _GATHER), jnp.int32),
            pltpu.VMEM((CH, D), jnp.float32),
            pltpu.SemaphoreType.DMA,
        ],
    )
    def k(idx_hbm, w_hbm, out_hbm, idxraw, idxadj, rows, sem):
        wid = lax.axis_index("s") * info.num_cores + lax.axis_index("c")
        base0 = wid * per_w
        iota = lax.iota(jnp.int32, _LANES)

        def body(c, carry):
            base = base0 + c * CH
            pltpu.sync_copy(idx_hbm.at[pl.ds(base, CH)], idxraw)
            for g in range(CH // _LANES):
                off = ((iota + (_LANES * g) % 3) % 3) * V
                v = idxraw[pl.ds(_LANES * g, _LANES)] + off
                p = _LANES * g
                idxadj[p // _GATHER, pl.ds(p % _GATHER, _LANES)] = v
            copies = [
                pltpu.async_copy(
                    w_hbm.at[idxadj.at[j]],
                    rows.at[pl.ds(j * _GATHER, _GATHER)],
                    sem,
                )
                for j in range(T)
            ]
            for cp in copies:
                cp.wait()
            pltpu.sync_copy(rows, out_hbm.at[pl.ds(base, CH)])
            return carry

        lax.fori_loop(0, n_chunks, body, 0)

    out = k(idx_flat, table)
    return out.reshape(B, L, T * D)
